# baseline (speedup 1.0000x reference)
import functools

import jax
import jax.numpy as jnp
from jax import lax
from jax.experimental import pallas as pl
from jax.experimental.pallas import tpu as pltpu
from jax.experimental.pallas import tpu_sc as plsc

VOCAB = 65
NEMB = 32
T = 8
BATCH = 16384
ROWS = BATCH * T
NW = 32
RPW = ROWS // NW
CH = 128
NCH = RPW // CH


def _table_body(tok_ref, pos_ref, w_ref, b_ref, tab_ref):
    x = tok_ref[...] + pos_ref[...]
    tab = jnp.dot(x, w_ref[...], preferred_element_type=jnp.float32) + b_ref[...]
    lane = lax.broadcasted_iota(jnp.int32, tab.shape, 1)
    valid = lane < VOCAB
    neg = jnp.full_like(tab, -jnp.inf)
    m = jnp.max(jnp.where(valid, tab, neg), axis=1, keepdims=True)
    s = jnp.sum(jnp.where(valid, jnp.exp(tab - m), 0.0), axis=1, keepdims=True)
    lse = m + jnp.log(s)
    tab_ref[...] = jnp.where(lane == VOCAB, lse, tab)


_sc_mesh = plsc.VectorSubcoreMesh(core_axis_name="c", subcore_axis_name="s")


@functools.partial(
    pl.kernel,
    out_type=(
        jax.ShapeDtypeStruct((ROWS, VOCAB), jnp.float32),
        jax.ShapeDtypeStruct((NW, 16), jnp.float32),
    ),
    mesh=_sc_mesh,
    compiler_params=pltpu.CompilerParams(
        needs_layout_passes=False, use_tc_tiling_on_sc=True),
    scratch_types=[
        pltpu.VMEM((VOCAB * T, 128), jnp.float32),
        pltpu.VMEM((RPW,), jnp.int32),
        pltpu.VMEM((RPW,), jnp.int32),
        pltpu.VMEM((RPW,), jnp.int32),
        pltpu.VMEM((CH, VOCAB), jnp.float32),
        pltpu.VMEM((CH, VOCAB), jnp.float32),
        pltpu.VMEM((16,), jnp.float32),
        pltpu.SemaphoreType.DMA,
        pltpu.SemaphoreType.DMA,
    ],
)
def _sc_gather(tab_hbm, idx_hbm, tgt_hbm, out_hbm, part_hbm,
               tab_v, idx_v, tgt_v, cidx_v, comp_a, comp_b,
               acc_v, wsem_a, wsem_b):
    wid = lax.axis_index("s") * 2 + lax.axis_index("c")
    base = wid * RPW
    pltpu.sync_copy(tab_hbm, tab_v)
    pltpu.sync_copy(idx_hbm.at[pl.ds(base, RPW)], idx_v)
    pltpu.sync_copy(tgt_hbm.at[pl.ds(base, RPW)], tgt_v)
    tpat = lax.iota(jnp.int32, 16) & (T - 1)

    def idx_body(c, carry):
        for j in range(CH // 16):
            off = c * CH + j * 16
            iv = idx_v[pl.ds(off, 16)]
            cidx_v[pl.ds(off, 16)] = iv * T + tpat
        return carry

    lax.fori_loop(0, NCH, idx_body, 0)

    col64 = jnp.full((16,), VOCAB - 1, jnp.int32)
    col_lse = jnp.full((16,), VOCAB, jnp.int32)
    lane16 = lax.iota(jnp.int32, 16)

    def _pack(c, comp_v, acc):
        def pack_rows(j, carry2):
            rowvec = cidx_v[pl.ds(c * CH + j * 16, 16)]
            for u in range(16):
                r = j * 16 + u
                row = rowvec[u]
                for k in range(4):
                    comp_v[r, pl.ds(k * 16, 16)] = tab_v[row, pl.ds(k * 16, 16)]
            return carry2

        lax.fori_loop(0, CH // 16, pack_rows, 0)
        for j in range(CH // 16):
            rows = lane16 + j * 16
            cv = cidx_v[pl.ds(c * CH + j * 16, 16)]
            vals = plsc.load_gather(tab_v, [cv, col64])
            plsc.store_scatter(comp_v, [rows, col64], vals)
            tv = tgt_v[pl.ds(c * CH + j * 16, 16)]
            lse = plsc.load_gather(tab_v, [cv, col_lse])
            hit = plsc.load_gather(tab_v, [cv, tv])
            acc = acc + (lse - hit)
        return acc

    def _wr(c, comp_v, wsem):
        return pltpu.make_async_copy(
            comp_v, out_hbm.at[pl.ds(base + c * CH, CH)], wsem)

    def dma_body(i, acc):
        c0 = 2 * i
        c1 = 2 * i + 1

        @pl.when(i > 0)
        def _():
            _wr(c0 - 2, comp_a, wsem_a).wait()

        acc = _pack(c0, comp_a, acc)
        _wr(c0, comp_a, wsem_a).start()

        @pl.when(i > 0)
        def _():
            _wr(c1 - 2, comp_b, wsem_b).wait()

        acc = _pack(c1, comp_b, acc)
        _wr(c1, comp_b, wsem_b).start()
        return acc

    acc = lax.fori_loop(0, NCH // 2, dma_body, jnp.zeros((16,), jnp.float32))
    acc_v[...] = acc
    pltpu.sync_copy(acc_v, part_hbm.at[wid])
    _wr(NCH - 2, comp_a, wsem_a).wait()
    _wr(NCH - 1, comp_b, wsem_b).wait()


def kernel(idx, targets, tok_emb, pos_emb, W, b):
    assert idx.shape == (BATCH, T) and tok_emb.shape == (VOCAB, NEMB)
    tok_rep = jnp.repeat(tok_emb, T, axis=0)
    pos_tile = jnp.tile(pos_emb, (VOCAB, 1))
    w_pad = jnp.pad(W, ((0, 0), (0, 128 - VOCAB)))
    b_pad = jnp.pad(b, (0, 128 - VOCAB)).reshape(1, 128)
    tab = pl.pallas_call(
        _table_body,
        out_shape=jax.ShapeDtypeStruct((VOCAB * T, 128), jnp.float32),
    )(tok_rep, pos_tile, w_pad, b_pad)
    logits2, parts = _sc_gather(tab, idx.reshape(-1), targets.reshape(-1))
    loss = jnp.sum(parts) * (1.0 / ROWS)
    return (logits2, loss)


# speedup vs baseline: 1.2740x; 1.2740x over previous
import functools

import jax
import jax.numpy as jnp
from jax import lax
from jax.experimental import pallas as pl
from jax.experimental.pallas import tpu as pltpu
from jax.experimental.pallas import tpu_sc as plsc

VOCAB = 65
NEMB = 32
T = 8
BATCH = 16384
ROWS = BATCH * T
NW = 32
RPW = ROWS // NW
CH = 128
NCH = RPW // CH


def _table_body(tok_ref, pos_ref, w_ref, b_ref, tab_ref):
    x = tok_ref[...] + pos_ref[...]
    tab = jnp.dot(x, w_ref[...], preferred_element_type=jnp.float32) + b_ref[...]
    lane = lax.broadcasted_iota(jnp.int32, tab.shape, 1)
    valid = lane < VOCAB
    neg = jnp.full_like(tab, -jnp.inf)
    m = jnp.max(jnp.where(valid, tab, neg), axis=1, keepdims=True)
    s = jnp.sum(jnp.where(valid, jnp.exp(tab - m), 0.0), axis=1, keepdims=True)
    lse = m + jnp.log(s)
    tab_ref[...] = jnp.where(lane == VOCAB, lse, tab)


_sc_mesh = plsc.VectorSubcoreMesh(core_axis_name="c", subcore_axis_name="s")


@functools.partial(
    pl.kernel,
    out_type=(
        jax.ShapeDtypeStruct((ROWS, VOCAB), jnp.float32),
        jax.ShapeDtypeStruct((NW, 16), jnp.float32),
    ),
    mesh=_sc_mesh,
    compiler_params=pltpu.CompilerParams(
        needs_layout_passes=False, use_tc_tiling_on_sc=True),
    scratch_types=[
        pltpu.VMEM((VOCAB * T, 128), jnp.float32),
        pltpu.VMEM((RPW,), jnp.int32),
        pltpu.VMEM((RPW,), jnp.int32),
        pltpu.VMEM((RPW,), jnp.int32),
        pltpu.VMEM((CH, VOCAB), jnp.float32),
        pltpu.VMEM((CH, VOCAB), jnp.float32),
        pltpu.VMEM((16,), jnp.float32),
        pltpu.SemaphoreType.DMA,
        pltpu.SemaphoreType.DMA,
    ],
)
def _sc_gather(tab_hbm, idx_hbm, tgt_hbm, out_hbm, part_hbm,
               tab_v, idx_v, tgt_v, cidx_v, comp_a, comp_b,
               acc_v, wsem_a, wsem_b):
    wid = lax.axis_index("s") * 2 + lax.axis_index("c")
    base = wid * RPW
    pltpu.sync_copy(tab_hbm, tab_v)
    pltpu.sync_copy(idx_hbm.at[pl.ds(base, RPW)], idx_v)
    pltpu.sync_copy(tgt_hbm.at[pl.ds(base, RPW)], tgt_v)
    tpat = lax.iota(jnp.int32, 16) & (T - 1)

    def idx_body(c, carry):
        for j in range(CH // 16):
            off = c * CH + j * 16
            iv = idx_v[pl.ds(off, 16)]
            cidx_v[pl.ds(off, 16)] = iv * T + tpat
        return carry

    lax.fori_loop(0, NCH, idx_body, 0)

    col64 = jnp.full((16,), VOCAB - 1, jnp.int32)
    col_lse = jnp.full((16,), VOCAB, jnp.int32)
    lane16 = lax.iota(jnp.int32, 16)

    def _pack(c, comp_v, acc):
        def pack_rows(j, carry2):
            rowvec = cidx_v[pl.ds(c * CH + j * 16, 16)]
            for u in range(0, 16, 2):
                r = j * 16 + u
                row0 = rowvec[u]
                row1 = rowvec[u + 1]
                a = [tab_v[row0, pl.ds(k * 16, 16)] for k in range(4)]
                bvals = [tab_v[row1, pl.ds(k * 16, 16)] for k in range(4)]
                for k in range(4):
                    comp_v[r, pl.ds(k * 16, 16)] = a[k]
                for k in range(4):
                    comp_v[r + 1, pl.ds(k * 16, 16)] = bvals[k]
            return carry2

        lax.fori_loop(0, CH // 16, pack_rows, 0)
        for j in range(CH // 16):
            rows = lane16 + j * 16
            cv = cidx_v[pl.ds(c * CH + j * 16, 16)]
            vals = plsc.load_gather(tab_v, [cv, col64])
            plsc.store_scatter(comp_v, [rows, col64], vals)
            tv = tgt_v[pl.ds(c * CH + j * 16, 16)]
            lse = plsc.load_gather(tab_v, [cv, col_lse])
            hit = plsc.load_gather(tab_v, [cv, tv])
            acc = acc + (lse - hit)
        return acc

    def _wr(c, comp_v, wsem):
        return pltpu.make_async_copy(
            comp_v, out_hbm.at[pl.ds(base + c * CH, CH)], wsem)

    def dma_body(i, acc):
        c0 = 2 * i
        c1 = 2 * i + 1

        @pl.when(i > 0)
        def _():
            _wr(c0 - 2, comp_a, wsem_a).wait()

        acc = _pack(c0, comp_a, acc)
        _wr(c0, comp_a, wsem_a).start()

        @pl.when(i > 0)
        def _():
            _wr(c1 - 2, comp_b, wsem_b).wait()

        acc = _pack(c1, comp_b, acc)
        _wr(c1, comp_b, wsem_b).start()
        return acc

    acc = lax.fori_loop(0, NCH // 2, dma_body, jnp.zeros((16,), jnp.float32))
    acc_v[...] = acc
    pltpu.sync_copy(acc_v, part_hbm.at[wid])
    _wr(NCH - 2, comp_a, wsem_a).wait()
    _wr(NCH - 1, comp_b, wsem_b).wait()


def kernel(idx, targets, tok_emb, pos_emb, W, b):
    assert idx.shape == (BATCH, T) and tok_emb.shape == (VOCAB, NEMB)
    tok_rep = jnp.repeat(tok_emb, T, axis=0)
    pos_tile = jnp.tile(pos_emb, (VOCAB, 1))
    w_pad = jnp.pad(W, ((0, 0), (0, 128 - VOCAB)))
    b_pad = jnp.pad(b, (0, 128 - VOCAB)).reshape(1, 128)
    tab = pl.pallas_call(
        _table_body,
        out_shape=jax.ShapeDtypeStruct((VOCAB * T, 128), jnp.float32),
    )(tok_rep, pos_tile, w_pad, b_pad)
    logits2, parts = _sc_gather(tab, idx.reshape(-1), targets.reshape(-1))
    loss = jnp.sum(parts) * (1.0 / ROWS)
    return (logits2, loss)


# speedup vs baseline: 1.2740x; 1.0000x over previous
import functools

import jax
import jax.numpy as jnp
from jax import lax
from jax.experimental import pallas as pl
from jax.experimental.pallas import tpu as pltpu
from jax.experimental.pallas import tpu_sc as plsc

VOCAB = 65
NEMB = 32
T = 8
BATCH = 16384
ROWS = BATCH * T
NW = 32
RPW = ROWS // NW
CH = 128
NCH = RPW // CH


def _table_body(tok_ref, pos_ref, w_ref, b_ref, tab_ref):
    x = tok_ref[...] + pos_ref[...]
    tab = jnp.dot(x, w_ref[...], preferred_element_type=jnp.float32) + b_ref[...]
    lane = lax.broadcasted_iota(jnp.int32, tab.shape, 1)
    valid = lane < VOCAB
    neg = jnp.full_like(tab, -jnp.inf)
    m = jnp.max(jnp.where(valid, tab, neg), axis=1, keepdims=True)
    s = jnp.sum(jnp.where(valid, jnp.exp(tab - m), 0.0), axis=1, keepdims=True)
    lse = m + jnp.log(s)
    tab_ref[...] = jnp.where(lane == VOCAB, lse, tab)


_sc_mesh = plsc.VectorSubcoreMesh(core_axis_name="c", subcore_axis_name="s")


@functools.partial(
    pl.kernel,
    out_type=(
        jax.ShapeDtypeStruct((ROWS, VOCAB), jnp.float32),
        jax.ShapeDtypeStruct((NW, 16), jnp.float32),
    ),
    mesh=_sc_mesh,
    compiler_params=pltpu.CompilerParams(
        needs_layout_passes=False, use_tc_tiling_on_sc=True),
    scratch_types=[
        pltpu.VMEM((VOCAB * T, 128), jnp.float32),
        pltpu.VMEM((RPW,), jnp.int32),
        pltpu.VMEM((RPW,), jnp.int32),
        pltpu.VMEM((RPW,), jnp.int32),
        pltpu.VMEM((CH, VOCAB), jnp.float32),
        pltpu.VMEM((CH, VOCAB), jnp.float32),
        pltpu.VMEM((16,), jnp.float32),
        pltpu.SemaphoreType.DMA,
        pltpu.SemaphoreType.DMA,
    ],
)
def _sc_gather(tab_hbm, idx_hbm, tgt_hbm, out_hbm, part_hbm,
               tab_v, idx_v, tgt_v, cidx_v, comp_a, comp_b,
               acc_v, wsem_a, wsem_b):
    wid = lax.axis_index("s") * 2 + lax.axis_index("c")
    base = wid * RPW
    pltpu.sync_copy(tab_hbm, tab_v)
    pltpu.sync_copy(idx_hbm.at[pl.ds(base, RPW)], idx_v)
    pltpu.sync_copy(tgt_hbm.at[pl.ds(base, RPW)], tgt_v)
    tpat = lax.iota(jnp.int32, 16) & (T - 1)

    def idx_body(c, carry):
        for j in range(CH // 16):
            off = c * CH + j * 16
            iv = idx_v[pl.ds(off, 16)]
            cidx_v[pl.ds(off, 16)] = iv * T + tpat
        return carry

    lax.fori_loop(0, NCH, idx_body, 0)

    col64 = jnp.full((16,), VOCAB - 1, jnp.int32)
    col_lse = jnp.full((16,), VOCAB, jnp.int32)
    lane16 = lax.iota(jnp.int32, 16)

    def _pack(c, comp_v, acc):
        def pack_rows(j, carry2):
            rowvec = cidx_v[pl.ds(c * CH + j * 16, 16)]
            prev = None
            for u in range(0, 16, 2):
                row0 = rowvec[u]
                row1 = rowvec[u + 1]
                cur = ([tab_v[row0, pl.ds(k * 16, 16)] for k in range(4)]
                       + [tab_v[row1, pl.ds(k * 16, 16)] for k in range(4)])
                if prev is not None:
                    r = j * 16 + u - 2
                    for k in range(4):
                        comp_v[r, pl.ds(k * 16, 16)] = prev[k]
                        comp_v[r + 1, pl.ds(k * 16, 16)] = prev[4 + k]
                prev = cur
            r = j * 16 + 14
            for k in range(4):
                comp_v[r, pl.ds(k * 16, 16)] = prev[k]
                comp_v[r + 1, pl.ds(k * 16, 16)] = prev[4 + k]
            return carry2

        lax.fori_loop(0, CH // 16, pack_rows, 0)
        for j in range(CH // 16):
            rows = lane16 + j * 16
            cv = cidx_v[pl.ds(c * CH + j * 16, 16)]
            vals = plsc.load_gather(tab_v, [cv, col64])
            plsc.store_scatter(comp_v, [rows, col64], vals)
            tv = tgt_v[pl.ds(c * CH + j * 16, 16)]
            lse = plsc.load_gather(tab_v, [cv, col_lse])
            hit = plsc.load_gather(tab_v, [cv, tv])
            acc = acc + (lse - hit)
        return acc

    def _wr(c, comp_v, wsem):
        return pltpu.make_async_copy(
            comp_v, out_hbm.at[pl.ds(base + c * CH, CH)], wsem)

    def dma_body(i, acc):
        c0 = 2 * i
        c1 = 2 * i + 1

        @pl.when(i > 0)
        def _():
            _wr(c0 - 2, comp_a, wsem_a).wait()

        acc = _pack(c0, comp_a, acc)
        _wr(c0, comp_a, wsem_a).start()

        @pl.when(i > 0)
        def _():
            _wr(c1 - 2, comp_b, wsem_b).wait()

        acc = _pack(c1, comp_b, acc)
        _wr(c1, comp_b, wsem_b).start()
        return acc

    acc = lax.fori_loop(0, NCH // 2, dma_body, jnp.zeros((16,), jnp.float32))
    acc_v[...] = acc
    pltpu.sync_copy(acc_v, part_hbm.at[wid])
    _wr(NCH - 2, comp_a, wsem_a).wait()
    _wr(NCH - 1, comp_b, wsem_b).wait()


def kernel(idx, targets, tok_emb, pos_emb, W, b):
    assert idx.shape == (BATCH, T) and tok_emb.shape == (VOCAB, NEMB)
    tok_rep = jnp.repeat(tok_emb, T, axis=0)
    pos_tile = jnp.tile(pos_emb, (VOCAB, 1))
    w_pad = jnp.pad(W, ((0, 0), (0, 128 - VOCAB)))
    b_pad = jnp.pad(b, (0, 128 - VOCAB)).reshape(1, 128)
    tab = pl.pallas_call(
        _table_body,
        out_shape=jax.ShapeDtypeStruct((VOCAB * T, 128), jnp.float32),
    )(tok_rep, pos_tile, w_pad, b_pad)
    logits2, parts = _sc_gather(tab, idx.reshape(-1), targets.reshape(-1))
    loss = jnp.sum(parts) * (1.0 / ROWS)
    return (logits2, loss)
